# baseline (speedup 1.0000x reference)
import functools

import jax
import jax.numpy as jnp
from jax.experimental import pallas as pl
from jax.experimental.pallas import tpu as pltpu


def _silu(v):
    return v * jax.nn.sigmoid(v)


def _c2f_moe_kernel(x_ref, cv1_w_ref, cv1_b_ref, cv2_w_ref, cv2_b_ref,
                    router_w_ref, router_b_ref, shared_w_ref, shared_b_ref,
                    experts_w_ref, experts_b_ref, out_ref):
    f32 = jnp.float32
    xb = x_ref[0]
    y = _silu(jnp.dot(cv1_w_ref[...], xb, preferred_element_type=f32)
              + cv1_b_ref[...])
    c = y.shape[0] // 2
    y2 = y[c:, :]

    hw = y2.shape[1]
    pooled = jnp.sum(y2, axis=1, keepdims=True) * (1.0 / hw)
    logits = (jnp.dot(router_w_ref[...], pooled, preferred_element_type=f32)
              + router_b_ref[...])
    lmax = jnp.max(logits)
    gate_w = 1.0 / jnp.sum(jnp.exp(logits - lmax))
    ids = jax.lax.broadcasted_iota(jnp.int32, logits.shape, 0)
    idx = jnp.min(jnp.where(logits >= lmax, ids, logits.shape[0]))

    bf = jnp.bfloat16
    y_bf = y.astype(bf)
    y2_bf = y_bf[c:, :]
    ew = experts_w_ref[idx].astype(bf)
    eb = experts_b_ref[idx]
    shared = _silu(jnp.dot(shared_w_ref[...].astype(bf), y2_bf,
                           preferred_element_type=f32) + shared_b_ref[...])
    routed = gate_w * _silu(jnp.dot(ew, y2_bf,
                                    preferred_element_type=f32) + eb)
    moe = shared + routed

    w_y = cv2_w_ref[:, :2 * c].astype(bf)
    w_m = (cv2_w_ref[:, 2 * c:3 * c] + cv2_w_ref[:, 3 * c:]).astype(bf)
    out = _silu(jnp.dot(w_y, y_bf, preferred_element_type=f32)
                + jnp.dot(w_m, moe.astype(bf), preferred_element_type=f32)
                + cv2_b_ref[...])
    out_ref[0] = out


@functools.partial(jax.jit, static_argnames=("interpret",))
def kernel(x, cv1_w, cv1_b, cv2_w, cv2_b, router_w, router_b,
           shared_w, shared_b, experts_w, experts_b, interpret=False):
    B, C1, H, W = x.shape
    HW = H * W
    O = cv2_w.shape[0]
    E, c, _ = experts_w.shape

    x3 = x.reshape(B, C1, HW)
    full = lambda a: pl.BlockSpec(a.shape, lambda i: (0,) * a.ndim)
    args = (
        x3,
        cv1_w, cv1_b.reshape(-1, 1),
        cv2_w, cv2_b.reshape(-1, 1),
        router_w, router_b.reshape(-1, 1),
        shared_w, shared_b.reshape(-1, 1),
        experts_w, experts_b.reshape(E, c, 1),
    )
    in_specs = [pl.BlockSpec((1, C1, HW), lambda i: (i, 0, 0))]
    in_specs += [full(a) for a in args[1:]]
    out = pl.pallas_call(
        _c2f_moe_kernel,
        grid=(B,),
        in_specs=in_specs,
        out_specs=pl.BlockSpec((1, O, HW), lambda i: (i, 0, 0)),
        out_shape=jax.ShapeDtypeStruct((B, O, HW), jnp.float32),
        compiler_params=pltpu.CompilerParams(
            dimension_semantics=("parallel",)),
        interpret=interpret,
    )(*args)
    return out.reshape(B, O, H, W)


# speedup vs baseline: 1.0300x; 1.0300x over previous
import functools

import jax
import jax.numpy as jnp
from jax.experimental import pallas as pl
from jax.experimental.pallas import tpu as pltpu


def _silu(v):
    return v * (0.5 * jnp.tanh(0.5 * v) + 0.5)


def _c2f_moe_kernel(x_ref, cv1_w_ref, cv1_b_ref, cv2_w_ref, cv2_b_ref,
                    router_w_ref, router_b_ref, shared_w_ref, shared_b_ref,
                    experts_w_ref, experts_b_ref, out_ref):
    f32 = jnp.float32
    xb = x_ref[0]
    y = _silu(jnp.dot(cv1_w_ref[...], xb, preferred_element_type=f32)
              + cv1_b_ref[...])
    c = y.shape[0] // 2
    y2 = y[c:, :]

    hw = y2.shape[1]
    pooled = jnp.sum(y2, axis=1, keepdims=True) * (1.0 / hw)
    logits = (jnp.dot(router_w_ref[...], pooled, preferred_element_type=f32)
              + router_b_ref[...])
    lmax = jnp.max(logits)
    gate_w = 1.0 / jnp.sum(jnp.exp(logits - lmax))
    ids = jax.lax.broadcasted_iota(jnp.int32, logits.shape, 0)
    idx = jnp.min(jnp.where(logits >= lmax, ids, logits.shape[0]))

    bf = jnp.bfloat16
    y_bf = y.astype(bf)
    y2_bf = y_bf[c:, :]
    ew = experts_w_ref[idx].astype(bf)
    eb = experts_b_ref[idx]
    shared = _silu(jnp.dot(shared_w_ref[...].astype(bf), y2_bf,
                           preferred_element_type=f32) + shared_b_ref[...])
    routed = gate_w * _silu(jnp.dot(ew, y2_bf,
                                    preferred_element_type=f32) + eb)
    moe = shared + routed

    w_y = cv2_w_ref[:, :2 * c].astype(bf)
    w_m = (cv2_w_ref[:, 2 * c:3 * c] + cv2_w_ref[:, 3 * c:]).astype(bf)
    out = _silu(jnp.dot(w_y, y_bf, preferred_element_type=f32)
                + jnp.dot(w_m, moe.astype(bf), preferred_element_type=f32)
                + cv2_b_ref[...])
    out_ref[0] = out


@functools.partial(jax.jit, static_argnames=("interpret",))
def kernel(x, cv1_w, cv1_b, cv2_w, cv2_b, router_w, router_b,
           shared_w, shared_b, experts_w, experts_b, interpret=False):
    B, C1, H, W = x.shape
    HW = H * W
    O = cv2_w.shape[0]
    E, c, _ = experts_w.shape

    x3 = x.reshape(B, C1, HW)
    full = lambda a: pl.BlockSpec(a.shape, lambda i: (0,) * a.ndim)
    args = (
        x3,
        cv1_w, cv1_b.reshape(-1, 1),
        cv2_w, cv2_b.reshape(-1, 1),
        router_w, router_b.reshape(-1, 1),
        shared_w, shared_b.reshape(-1, 1),
        experts_w, experts_b.reshape(E, c, 1),
    )
    in_specs = [pl.BlockSpec((1, C1, HW), lambda i: (i, 0, 0))]
    in_specs += [full(a) for a in args[1:]]
    out = pl.pallas_call(
        _c2f_moe_kernel,
        grid=(B,),
        in_specs=in_specs,
        out_specs=pl.BlockSpec((1, O, HW), lambda i: (i, 0, 0)),
        out_shape=jax.ShapeDtypeStruct((B, O, HW), jnp.float32),
        compiler_params=pltpu.CompilerParams(
            dimension_semantics=("parallel",)),
        interpret=interpret,
    )(*args)
    return out.reshape(B, O, H, W)


# speedup vs baseline: 2.5691x; 2.4942x over previous
import functools

import jax
import jax.numpy as jnp
from jax.experimental import pallas as pl
from jax.experimental.pallas import tpu as pltpu


def _silu(v):
    return v * (0.5 * jnp.tanh(0.5 * v) + 0.5)


def _dot_t(a, b):
    return jax.lax.dot_general(a, b, (((1,), (1,)), ((), ())),
                               preferred_element_type=jnp.float32)


def _c2f_moe_kernel(x_ref, cv1_w_ref, cv1_b_ref, cv2_w_ref, cv2_b_ref,
                    router_w_ref, router_b_ref, shared_w_ref, shared_b_ref,
                    experts_w_ref, experts_b_ref, out_ref):
    bf = jnp.bfloat16
    xb = x_ref[0]
    y = _silu(_dot_t(xb, cv1_w_ref[...]) + cv1_b_ref[...])
    c = y.shape[1] // 2
    y2 = y[:, c:]

    hw = y2.shape[0]
    pooled = jnp.sum(y2, axis=0, keepdims=True) * (1.0 / hw)
    logits = _dot_t(pooled, router_w_ref[...]) + router_b_ref[...]
    lmax = jnp.max(logits)
    gate_w = 1.0 / jnp.sum(jnp.exp(logits - lmax))
    ids = jax.lax.broadcasted_iota(jnp.int32, logits.shape, 1)
    idx = jnp.min(jnp.where(logits >= lmax, ids, logits.shape[1]))

    y2_bf = y2.astype(bf)
    ew = experts_w_ref[idx].astype(bf)
    eb = experts_b_ref[idx]
    shared = _silu(_dot_t(y2_bf, shared_w_ref[...].astype(bf))
                   + shared_b_ref[...])
    routed = gate_w * _silu(_dot_t(y2_bf, ew) + eb)
    moe = shared + routed

    w_y = cv2_w_ref[:, :2 * c].astype(bf)
    w_m = (cv2_w_ref[:, 2 * c:3 * c] + cv2_w_ref[:, 3 * c:]).astype(bf)
    out = _silu(_dot_t(y.astype(bf), w_y)
                + _dot_t(moe.astype(bf), w_m)
                + cv2_b_ref[...])
    out_ref[0] = out


@functools.partial(jax.jit, static_argnames=("interpret",))
def kernel(x, cv1_w, cv1_b, cv2_w, cv2_b, router_w, router_b,
           shared_w, shared_b, experts_w, experts_b, interpret=False):
    B, C1, H, W = x.shape
    HW = H * W
    O = cv2_w.shape[0]
    E, c, _ = experts_w.shape

    xt = x.transpose(0, 2, 3, 1).reshape(B, HW, C1)
    full = lambda a: pl.BlockSpec(a.shape, lambda i: (0,) * a.ndim)
    args = (
        xt,
        cv1_w, cv1_b.reshape(1, -1),
        cv2_w, cv2_b.reshape(1, -1),
        router_w, router_b.reshape(1, -1),
        shared_w, shared_b.reshape(1, -1),
        experts_w, experts_b.reshape(E, 1, c),
    )
    in_specs = [pl.BlockSpec((1, HW, C1), lambda i: (i, 0, 0))]
    in_specs += [full(a) for a in args[1:]]
    out = pl.pallas_call(
        _c2f_moe_kernel,
        grid=(B,),
        in_specs=in_specs,
        out_specs=pl.BlockSpec((1, HW, O), lambda i: (i, 0, 0)),
        out_shape=jax.ShapeDtypeStruct((B, HW, O), jnp.float32),
        compiler_params=pltpu.CompilerParams(
            dimension_semantics=("parallel",)),
        interpret=interpret,
    )(*args)
    return out.reshape(B, H, W, O).transpose(0, 3, 1, 2)


# speedup vs baseline: 2.6115x; 1.0165x over previous
import functools

import jax
import jax.numpy as jnp
from jax.experimental import pallas as pl
from jax.experimental.pallas import tpu as pltpu


def _silu(v):
    u = 0.5 * v
    return u + u * jnp.tanh(u)


def _dot_t(a, b, precision=None):
    return jax.lax.dot_general(a, b, (((1,), (1,)), ((), ())),
                               preferred_element_type=jnp.float32,
                               precision=precision)


def _c2f_moe_kernel(x_ref, cv1_w_ref, cv1_b_ref, cv2_w_ref, cv2_b_ref,
                    router_w_ref, router_b_ref, shared_w_ref, shared_b_ref,
                    experts_w_ref, experts_b_ref, out_ref):
    bf = jnp.bfloat16
    xb = x_ref[0]
    y = _silu(_dot_t(xb, cv1_w_ref[...]) + cv1_b_ref[...])
    c = y.shape[1] // 2
    y2 = y[:, c:]

    hw = y2.shape[0]
    pooled = jnp.sum(y2, axis=0, keepdims=True) * (1.0 / hw)
    logits = _dot_t(pooled, router_w_ref[...]) + router_b_ref[...]
    lmax = jnp.max(logits)
    gate_w = 1.0 / jnp.sum(jnp.exp(logits - lmax))
    ids = jax.lax.broadcasted_iota(jnp.int32, logits.shape, 1)
    idx = jnp.min(jnp.where(logits >= lmax, ids, logits.shape[1]))

    y2_bf = y2.astype(bf)
    ew = experts_w_ref[idx].astype(bf)
    eb = experts_b_ref[idx]
    shared = _silu(_dot_t(y2_bf, shared_w_ref[...].astype(bf))
                   + shared_b_ref[...])
    routed = gate_w * _silu(_dot_t(y2_bf, ew) + eb)
    moe = shared + routed

    w_y = cv2_w_ref[:, :2 * c].astype(bf)
    w_m = (cv2_w_ref[:, 2 * c:3 * c] + cv2_w_ref[:, 3 * c:]).astype(bf)
    out = _silu(_dot_t(y.astype(bf), w_y)
                + _dot_t(moe.astype(bf), w_m)
                + cv2_b_ref[...])
    out_ref[0] = out


@functools.partial(jax.jit, static_argnames=("interpret",))
def kernel(x, cv1_w, cv1_b, cv2_w, cv2_b, router_w, router_b,
           shared_w, shared_b, experts_w, experts_b, interpret=False):
    B, C1, H, W = x.shape
    HW = H * W
    O = cv2_w.shape[0]
    E, c, _ = experts_w.shape

    xt = x.transpose(0, 2, 3, 1).reshape(B, HW, C1)
    full = lambda a: pl.BlockSpec(a.shape, lambda i: (0,) * a.ndim)
    args = (
        xt,
        cv1_w, cv1_b.reshape(1, -1),
        cv2_w, cv2_b.reshape(1, -1),
        router_w, router_b.reshape(1, -1),
        shared_w, shared_b.reshape(1, -1),
        experts_w, experts_b.reshape(E, 1, c),
    )
    in_specs = [pl.BlockSpec((1, HW, C1), lambda i: (i, 0, 0))]
    in_specs += [full(a) for a in args[1:]]
    out = pl.pallas_call(
        _c2f_moe_kernel,
        grid=(B,),
        in_specs=in_specs,
        out_specs=pl.BlockSpec((1, HW, O), lambda i: (i, 0, 0)),
        out_shape=jax.ShapeDtypeStruct((B, HW, O), jnp.float32),
        compiler_params=pltpu.CompilerParams(
            dimension_semantics=("parallel",)),
        interpret=interpret,
    )(*args)
    return out.reshape(B, H, W, O).transpose(0, 3, 1, 2)


# speedup vs baseline: 2.8311x; 1.0841x over previous
import functools

import jax
import jax.numpy as jnp
from jax.experimental import pallas as pl
from jax.experimental.pallas import tpu as pltpu


def _silu(v):
    u = 0.5 * v
    return u + u * jnp.tanh(u)


def _dot_t(a, b):
    return jax.lax.dot_general(a, b, (((1,), (1,)), ((), ())),
                               preferred_element_type=jnp.float32)


def _c2f_moe_kernel(x_ref, cv1_w_ref, cv1_b_ref, cv2_w_ref, cv2_b_ref,
                    router_w_ref, router_b_ref, shared_w_ref, shared_b_ref,
                    experts_w_ref, experts_b_ref, out_ref):
    bf = jnp.bfloat16

    def one_image(xb):
        y = _silu(_dot_t(xb, cv1_w_ref[...]) + cv1_b_ref[...])
        c = y.shape[1] // 2
        y2 = y[:, c:]

        hw = y2.shape[0]
        pooled = jnp.sum(y2, axis=0, keepdims=True) * (1.0 / hw)
        logits = _dot_t(pooled, router_w_ref[...]) + router_b_ref[...]
        lmax = jnp.max(logits)
        gate_w = 1.0 / jnp.sum(jnp.exp(logits - lmax))
        ids = jax.lax.broadcasted_iota(jnp.int32, logits.shape, 1)
        idx = jnp.min(jnp.where(logits >= lmax, ids, logits.shape[1]))

        y_bf = y.astype(bf)
        y2_bf = y_bf[:, c:]
        ew = experts_w_ref[idx].astype(bf)
        eb = experts_b_ref[idx].astype(bf)
        shared = _silu(_dot_t(y2_bf, shared_w_ref[...].astype(bf)).astype(bf)
                       + shared_b_ref[...].astype(bf))
        routed = gate_w.astype(bf) * _silu(_dot_t(y2_bf, ew).astype(bf) + eb)
        moe = shared + routed

        w_y = cv2_w_ref[:, :2 * c].astype(bf)
        w_m = (cv2_w_ref[:, 2 * c:3 * c] + cv2_w_ref[:, 3 * c:]).astype(bf)
        return _silu(_dot_t(y_bf, w_y) + _dot_t(moe, w_m) + cv2_b_ref[...])

    for j in range(x_ref.shape[0]):
        out_ref[j] = one_image(x_ref[j])


@functools.partial(jax.jit, static_argnames=("interpret",))
def kernel(x, cv1_w, cv1_b, cv2_w, cv2_b, router_w, router_b,
           shared_w, shared_b, experts_w, experts_b, interpret=False):
    B, C1, H, W = x.shape
    HW = H * W
    O = cv2_w.shape[0]
    E, c, _ = experts_w.shape
    PAIR = 2

    xt = x.transpose(0, 2, 3, 1).reshape(B, HW, C1)
    full = lambda a: pl.BlockSpec(a.shape, lambda i: (0,) * a.ndim)
    args = (
        xt,
        cv1_w, cv1_b.reshape(1, -1),
        cv2_w, cv2_b.reshape(1, -1),
        router_w, router_b.reshape(1, -1),
        shared_w, shared_b.reshape(1, -1),
        experts_w, experts_b.reshape(E, 1, c),
    )
    in_specs = [pl.BlockSpec((PAIR, HW, C1), lambda i: (i, 0, 0))]
    in_specs += [full(a) for a in args[1:]]
    out = pl.pallas_call(
        _c2f_moe_kernel,
        grid=(B // PAIR,),
        in_specs=in_specs,
        out_specs=pl.BlockSpec((PAIR, HW, O), lambda i: (i, 0, 0)),
        out_shape=jax.ShapeDtypeStruct((B, HW, O), jnp.float32),
        compiler_params=pltpu.CompilerParams(
            dimension_semantics=("parallel",)),
        interpret=interpret,
    )(*args)
    return out.reshape(B, H, W, O).transpose(0, 3, 1, 2)


# speedup vs baseline: 2.9544x; 1.0436x over previous
import functools

import jax
import jax.numpy as jnp
from jax.experimental import pallas as pl
from jax.experimental.pallas import tpu as pltpu


def _silu(v):
    u = 0.5 * v
    return u + u * jnp.tanh(u)


def _dot_t(a, b):
    return jax.lax.dot_general(a, b, (((1,), (1,)), ((), ())),
                               preferred_element_type=jnp.float32)


def _c2f_moe_kernel(x_ref, cv1_w_ref, cv1_b_ref, cv2_w_ref, cv2_b_ref,
                    router_w_ref, router_b_ref, shared_w_ref, shared_b_ref,
                    experts_w_ref, experts_b_ref, out_ref):
    bf = jnp.bfloat16

    def one_image(xb):
        y = _silu(_dot_t(xb, cv1_w_ref[...]) + cv1_b_ref[...])
        c = y.shape[1] // 2
        y2 = y[:, c:]

        hw = y2.shape[0]
        pooled = jnp.sum(y2, axis=0, keepdims=True) * (1.0 / hw)
        logits = _dot_t(pooled, router_w_ref[...]) + router_b_ref[...]
        lmax = jnp.max(logits)
        gate_w = 1.0 / jnp.sum(jnp.exp(logits - lmax))
        ids = jax.lax.broadcasted_iota(jnp.int32, logits.shape, 1)
        idx = jnp.min(jnp.where(logits >= lmax, ids, logits.shape[1]))

        y_bf = y.astype(bf)
        y2_bf = y_bf[:, c:]
        ew = experts_w_ref[idx].astype(bf)
        eb = experts_b_ref[pl.ds(idx, 1), :].astype(bf)
        shared = _silu(_dot_t(y2_bf, shared_w_ref[...].astype(bf)).astype(bf)
                       + shared_b_ref[...].astype(bf))
        routed = gate_w.astype(bf) * _silu(_dot_t(y2_bf, ew).astype(bf) + eb)
        moe = shared + routed

        w_y = cv2_w_ref[:, :2 * c].astype(bf)
        w_m = (cv2_w_ref[:, 2 * c:3 * c] + cv2_w_ref[:, 3 * c:]).astype(bf)
        return _silu(_dot_t(y_bf, w_y) + _dot_t(moe, w_m) + cv2_b_ref[...])

    for j in range(x_ref.shape[0]):
        out_ref[j] = one_image(x_ref[j])


@functools.partial(jax.jit, static_argnames=("interpret",))
def kernel(x, cv1_w, cv1_b, cv2_w, cv2_b, router_w, router_b,
           shared_w, shared_b, experts_w, experts_b, interpret=False):
    B, C1, H, W = x.shape
    HW = H * W
    O = cv2_w.shape[0]
    E, c, _ = experts_w.shape
    PAIR = 2

    xt = x.transpose(0, 2, 3, 1).reshape(B, HW, C1)
    full = lambda a: pl.BlockSpec(a.shape, lambda i: (0,) * a.ndim)
    args = (
        xt,
        cv1_w, cv1_b.reshape(1, -1),
        cv2_w, cv2_b.reshape(1, -1),
        router_w, router_b.reshape(1, -1),
        shared_w, shared_b.reshape(1, -1),
        experts_w, experts_b,
    )
    in_specs = [pl.BlockSpec((PAIR, HW, C1), lambda i: (i, 0, 0))]
    in_specs += [full(a) for a in args[1:]]
    out = pl.pallas_call(
        _c2f_moe_kernel,
        grid=(B // PAIR,),
        in_specs=in_specs,
        out_specs=pl.BlockSpec((PAIR, HW, O), lambda i: (i, 0, 0)),
        out_shape=jax.ShapeDtypeStruct((B, HW, O), jnp.float32),
        compiler_params=pltpu.CompilerParams(
            dimension_semantics=("parallel",)),
        interpret=interpret,
    )(*args)
    return out.reshape(B, H, W, O).transpose(0, 3, 1, 2)


# speedup vs baseline: 2.9780x; 1.0080x over previous
import jax
import jax.numpy as jnp
from jax.experimental import pallas as pl
from jax.experimental.pallas import tpu as pltpu


def _silu(v):
    u = 0.5 * v
    return u + u * jnp.tanh(u)


def _dot_t(a, b):
    return jax.lax.dot_general(a, b, (((1,), (1,)), ((), ())),
                               preferred_element_type=jnp.float32)


def _c2f_moe_kernel(x_ref, cv1_w_ref, cv1_b_ref, cv2_w_ref, cv2_b_ref,
                    router_w_ref, router_b_ref, shared_w_ref, shared_b_ref,
                    experts_w_ref, experts_b_ref, out_ref):
    bf = jnp.bfloat16

    def one_image(xb):
        y = _silu(_dot_t(xb, cv1_w_ref[...]) + cv1_b_ref[...])
        c = y.shape[1] // 2
        y2 = y[:, c:]

        hw = y2.shape[0]
        pooled = jnp.sum(y2, axis=0, keepdims=True) * (1.0 / hw)
        logits = _dot_t(pooled, router_w_ref[...]) + router_b_ref[...]
        lmax = jnp.max(logits)
        gate_w = 1.0 / jnp.sum(jnp.exp(logits - lmax))
        ids = jax.lax.broadcasted_iota(jnp.int32, logits.shape, 1)
        idx = jnp.min(jnp.where(logits >= lmax, ids, logits.shape[1]))

        y_bf = y.astype(bf)
        y2_bf = y_bf[:, c:]
        ew = experts_w_ref[idx].astype(bf)
        eb = experts_b_ref[pl.ds(idx, 1), :].astype(bf)
        shared = _silu(_dot_t(y2_bf, shared_w_ref[...].astype(bf)).astype(bf)
                       + shared_b_ref[...].astype(bf))
        routed = gate_w.astype(bf) * _silu(_dot_t(y2_bf, ew).astype(bf) + eb)
        moe = shared + routed

        w_y = cv2_w_ref[:, :2 * c].astype(bf)
        w_m = (cv2_w_ref[:, 2 * c:3 * c] + cv2_w_ref[:, 3 * c:]).astype(bf)
        return _silu(_dot_t(y_bf, w_y) + _dot_t(moe, w_m) + cv2_b_ref[...])

    for j in range(x_ref.shape[0]):
        out_ref[j] = one_image(x_ref[j])


@jax.jit
def kernel(x, cv1_w, cv1_b, cv2_w, cv2_b, router_w, router_b,
           shared_w, shared_b, experts_w, experts_b):
    B, C1, H, W = x.shape
    HW = H * W
    O = cv2_w.shape[0]
    E, c, _ = experts_w.shape
    PAIR = 2

    xt = x.transpose(0, 2, 3, 1).reshape(B, HW, C1)
    full = lambda a: pl.BlockSpec(a.shape, lambda i: (0,) * a.ndim)
    args = (
        xt,
        cv1_w, cv1_b.reshape(1, -1),
        cv2_w, cv2_b.reshape(1, -1),
        router_w, router_b.reshape(1, -1),
        shared_w, shared_b.reshape(1, -1),
        experts_w, experts_b,
    )
    in_specs = [pl.BlockSpec((PAIR, HW, C1), lambda i: (i, 0, 0))]
    in_specs += [full(a) for a in args[1:]]
    out = pl.pallas_call(
        _c2f_moe_kernel,
        grid=(B // PAIR,),
        in_specs=in_specs,
        out_specs=pl.BlockSpec((PAIR, HW, O), lambda i: (i, 0, 0)),
        out_shape=jax.ShapeDtypeStruct((B, HW, O), jnp.float32),
        compiler_params=pltpu.CompilerParams(
            dimension_semantics=("parallel",)),
    )(*args)
    return out.reshape(B, H, W, O).transpose(0, 3, 1, 2)
